# baseline (speedup 1.0000x reference)
import jax
import jax.numpy as jnp
from jax.experimental import pallas as pl

SZ = 2048
TEMP = 10.0
EPS = 1e-20
ROWS_PER_BLOCK = 512


def _native_view(x):
    return (
        x.reshape(SZ, 16, 128, 2)
        .transpose(0, 1, 3, 2)
        .reshape(SZ, 32, 128)
    )


def _body(g_ref, u_ref, o_ref):
    for g in range(16):
        a0 = g_ref[:, 2 * g, :]
        a1 = g_ref[:, 2 * g + 1, :]
        u0 = u_ref[:, 2 * g, :]
        u1 = u_ref[:, 2 * g + 1, :]
        L0 = EPS - jnp.log(u0 + EPS)
        L1 = EPS - jnp.log(u1 + EPS)
        lr = jnp.log(L0 / L1)
        s = (a0 - a1 - lr) * (1.0 / TEMP)
        o_ref[:, 128 * g:128 * (g + 1)] = 1.0 / (1.0 + jnp.exp(-s))


def kernel(gen_matrix, u):
    gm = _native_view(gen_matrix.reshape(SZ, SZ, 2))
    uu = _native_view(u.reshape(SZ, SZ, 2))
    grid = SZ // ROWS_PER_BLOCK
    return pl.pallas_call(
        _body,
        grid=(grid,),
        in_specs=[
            pl.BlockSpec((ROWS_PER_BLOCK, 32, 128), lambda i: (i, 0, 0)),
            pl.BlockSpec((ROWS_PER_BLOCK, 32, 128), lambda i: (i, 0, 0)),
        ],
        out_specs=pl.BlockSpec((ROWS_PER_BLOCK, SZ), lambda i: (i, 0)),
        out_shape=jax.ShapeDtypeStruct((SZ, SZ), jnp.float32),
    )(gm, uu)


# speedup vs baseline: 1.0362x; 1.0362x over previous
import jax
import jax.numpy as jnp
from jax.experimental import pallas as pl

SZ = 2048
TEMP = 10.0
EPS = 1e-20
ROWS_PER_BLOCK = 128


def _native_view(x):
    return (
        x.reshape(SZ, 16, 128, 2)
        .transpose(0, 1, 3, 2)
        .reshape(SZ, 32, 128)
    )


def _body(g_ref, u_ref, o_ref):
    for g in range(16):
        a0 = g_ref[:, 2 * g, :]
        a1 = g_ref[:, 2 * g + 1, :]
        u0 = u_ref[:, 2 * g, :]
        u1 = u_ref[:, 2 * g + 1, :]
        L0 = EPS - jnp.log(u0 + EPS)
        L1 = EPS - jnp.log(u1 + EPS)
        lr = jnp.log(L0 / L1)
        s = (a0 - a1 - lr) * (1.0 / TEMP)
        o_ref[:, 128 * g:128 * (g + 1)] = 1.0 / (1.0 + jnp.exp(-s))


def kernel(gen_matrix, u):
    gm = _native_view(gen_matrix.reshape(SZ, SZ, 2))
    uu = _native_view(u.reshape(SZ, SZ, 2))
    grid = SZ // ROWS_PER_BLOCK
    return pl.pallas_call(
        _body,
        grid=(grid,),
        in_specs=[
            pl.BlockSpec((ROWS_PER_BLOCK, 32, 128), lambda i: (i, 0, 0)),
            pl.BlockSpec((ROWS_PER_BLOCK, 32, 128), lambda i: (i, 0, 0)),
        ],
        out_specs=pl.BlockSpec((ROWS_PER_BLOCK, SZ), lambda i: (i, 0)),
        out_shape=jax.ShapeDtypeStruct((SZ, SZ), jnp.float32),
    )(gm, uu)


# speedup vs baseline: 1.0373x; 1.0011x over previous
import jax
import jax.numpy as jnp
from jax.experimental import pallas as pl

SZ = 2048
TEMP = 10.0
EPS = 1e-20
ROWS_PER_BLOCK = 64


def _native_view(x):
    return (
        x.reshape(SZ, 16, 128, 2)
        .transpose(0, 1, 3, 2)
        .reshape(SZ, 32, 128)
    )


def _body(g_ref, u_ref, o_ref):
    for g in range(16):
        a0 = g_ref[:, 2 * g, :]
        a1 = g_ref[:, 2 * g + 1, :]
        u0 = u_ref[:, 2 * g, :]
        u1 = u_ref[:, 2 * g + 1, :]
        L0 = EPS - jnp.log(u0 + EPS)
        L1 = EPS - jnp.log(u1 + EPS)
        lr = jnp.log(L0 / L1)
        s = (a0 - a1 - lr) * (1.0 / TEMP)
        o_ref[:, 128 * g:128 * (g + 1)] = 1.0 / (1.0 + jnp.exp(-s))


def kernel(gen_matrix, u):
    gm = _native_view(gen_matrix.reshape(SZ, SZ, 2))
    uu = _native_view(u.reshape(SZ, SZ, 2))
    grid = SZ // ROWS_PER_BLOCK
    return pl.pallas_call(
        _body,
        grid=(grid,),
        in_specs=[
            pl.BlockSpec((ROWS_PER_BLOCK, 32, 128), lambda i: (i, 0, 0)),
            pl.BlockSpec((ROWS_PER_BLOCK, 32, 128), lambda i: (i, 0, 0)),
        ],
        out_specs=pl.BlockSpec((ROWS_PER_BLOCK, SZ), lambda i: (i, 0)),
        out_shape=jax.ShapeDtypeStruct((SZ, SZ), jnp.float32),
    )(gm, uu)
